# Initial kernel scaffold; baseline (speedup 1.0000x reference)
import jax
import jax.numpy as jnp
from jax.experimental import pallas as pl


def kernel(x, edge_index, W1, b1, W2, b2):
    raise NotImplementedError("write your pallas kernel here")


# speedup vs baseline: 33.9451x; 33.9451x over previous
import functools

import jax
import jax.numpy as jnp
from jax import lax
from jax.experimental import pallas as pl
from jax.experimental.pallas import tpu as pltpu
from jax.experimental.pallas import tpu_sc as plsc

N = 10000
D = 128
H = 16
C = 2
E = 320000

NC = 2
NS = 16
NW = NC * NS
CHUNK = 128
NB = 4
CH = 80
E_PAD = NW * CH * CHUNK
NP = 10240
RPS = NP // NS


def _sc_mesh():
    return plsc.VectorSubcoreMesh(
        core_axis_name="c", subcore_axis_name="s", num_cores=NC, num_subcores=NS
    )


def _deg_body(dstw, out, didx, ones, zbuf, acc, ssem):
    c = lax.axis_index("c")
    s = lax.axis_index("s")
    w = c * NS + s

    def fill(i, _):
        ones[pl.ds(i * 16, 16)] = jnp.ones((16,), jnp.float32)
        zbuf[pl.ds(i * 16, 16)] = jnp.zeros((16,), jnp.float32)
        return 0

    lax.fori_loop(0, RPS // 16, fill, 0)
    pltpu.sync_copy(zbuf, acc.at[pl.ds(s * RPS, RPS)])
    plsc.subcore_barrier()

    pltpu.sync_copy(dstw.at[w], didx)

    def group(g, _):
        descs = [
            pltpu.async_copy(
                ones, acc.at[didx.at[g * NB + b]], ssem.at[b], add=True
            )
            for b in range(NB)
        ]
        for d in descs:
            d.wait()
        return 0

    lax.fori_loop(0, CH // NB, group, 0)
    plsc.subcore_barrier()
    pltpu.sync_copy(acc.at[pl.ds(s * RPS, RPS)], out.at[c, pl.ds(s * RPS, RPS)])


def _agg_body(table, srcw, dstw, out, sidx, didx, rows, zbuf, acc, gsem, ssem):
    c = lax.axis_index("c")
    s = lax.axis_index("s")
    w = c * NS + s

    def zf(i, _):
        zbuf[i] = jnp.zeros((H,), jnp.float32)
        return 0

    lax.fori_loop(0, RPS, zf, 0)
    pltpu.sync_copy(zbuf, acc.at[pl.ds(s * RPS, RPS)])
    plsc.subcore_barrier()

    pltpu.sync_copy(srcw.at[w], sidx)
    pltpu.sync_copy(dstw.at[w], didx)

    def group(g, _):
        gds = [
            pltpu.async_copy(table.at[sidx.at[g * NB + b]], rows.at[b], gsem.at[b])
            for b in range(NB)
        ]
        sds = []
        for b in range(NB):
            gds[b].wait()
            sds.append(
                pltpu.async_copy(
                    rows.at[b], acc.at[didx.at[g * NB + b]], ssem.at[b], add=True
                )
            )
        for d in sds:
            d.wait()
        return 0

    lax.fori_loop(0, CH // NB, group, 0)
    plsc.subcore_barrier()
    pltpu.sync_copy(acc.at[pl.ds(s * RPS, RPS)], out.at[c, pl.ds(s * RPS, RPS)])


@functools.lru_cache(maxsize=1)
def _sc_kernels():
    params = pltpu.CompilerParams(use_tc_tiling_on_sc=False)
    deg = pl.kernel(
        _deg_body,
        out_type=jax.ShapeDtypeStruct((NC, NP), jnp.float32),
        mesh=_sc_mesh(),
        compiler_params=params,
        scratch_types=[
            pltpu.VMEM((CH, CHUNK), jnp.int32),
            pltpu.VMEM((CHUNK,), jnp.float32),
            pltpu.VMEM((RPS,), jnp.float32),
            pltpu.VMEM_SHARED((NP,), jnp.float32),
            pltpu.SemaphoreType.DMA((NB,)),
        ],
    )
    agg = pl.kernel(
        _agg_body,
        out_type=jax.ShapeDtypeStruct((NC, NP, H), jnp.float32),
        mesh=_sc_mesh(),
        compiler_params=params,
        scratch_types=[
            pltpu.VMEM((CH, CHUNK), jnp.int32),
            pltpu.VMEM((CH, CHUNK), jnp.int32),
            pltpu.VMEM((NB, CHUNK, H), jnp.float32),
            pltpu.VMEM((RPS, H), jnp.float32),
            pltpu.VMEM_SHARED((NP, H), jnp.float32),
            pltpu.SemaphoreType.DMA((NB,)),
            pltpu.SemaphoreType.DMA((NB,)),
        ],
    )
    return deg, agg


def _dinv(dp):
    return lax.rsqrt(dp[:, 0:1] + dp[:, 1:2] + 1.0)[:N]


def _tc1_body(x_ref, w1_ref, dp_ref, y1_ref):
    dinv = _dinv(dp_ref[...])
    xw = jnp.dot(x_ref[...], w1_ref[...], preferred_element_type=jnp.float32)
    y1_ref[...] = xw * dinv


def _tc2_body(a0_ref, a1_ref, y1_ref, dp_ref, b1_ref, y2_ref):
    dinv = _dinv(dp_ref[...])
    agg = a0_ref[...][:N] + a1_ref[...][:N] + y1_ref[...]
    h1 = jnp.maximum(agg * dinv + b1_ref[...], 0.0)
    y2_ref[...] = h1 * dinv


def _tc3_body(a0_ref, a1_ref, y2_ref, dp_ref, w2_ref, b2_ref, o_ref):
    dinv = _dinv(dp_ref[...])
    z = (a0_ref[...][:N] + a1_ref[...][:N] + y2_ref[...]) * dinv
    logits = (
        jnp.dot(z, w2_ref[...], preferred_element_type=jnp.float32) + b2_ref[...]
    )
    m = jnp.max(logits, axis=1, keepdims=True)
    lse = m + jnp.log(jnp.sum(jnp.exp(logits - m), axis=1, keepdims=True))
    o_ref[...] = logits - lse


def kernel(x, edge_index, W1, b1, W2, b2):
    src = edge_index[0]
    dst = edge_index[1]
    pad = E_PAD - E
    srcp = jnp.concatenate([src, jnp.zeros((pad,), jnp.int32)]).reshape(
        NW, CH, CHUNK
    )
    dstp = jnp.concatenate([dst, jnp.full((pad,), N, jnp.int32)]).reshape(
        NW, CH, CHUNK
    )

    deg_kernel, agg_kernel = _sc_kernels()
    degp = deg_kernel(dstp)
    dp2 = degp.T

    y1 = pl.pallas_call(
        _tc1_body, out_shape=jax.ShapeDtypeStruct((N, H), jnp.float32)
    )(x, W1, dp2)

    ap1 = agg_kernel(y1, srcp, dstp)
    y2 = pl.pallas_call(
        _tc2_body, out_shape=jax.ShapeDtypeStruct((N, H), jnp.float32)
    )(ap1[0], ap1[1], y1, dp2, b1.reshape(1, H))

    ap2 = agg_kernel(y2, srcp, dstp)
    out = pl.pallas_call(
        _tc3_body, out_shape=jax.ShapeDtypeStruct((N, C), jnp.float32)
    )(ap2[0], ap2[1], y2, dp2, W2, b2.reshape(1, C))
    return out


# speedup vs baseline: 36.8574x; 1.0858x over previous
import functools

import jax
import jax.numpy as jnp
from jax import lax
from jax.experimental import pallas as pl
from jax.experimental.pallas import tpu as pltpu
from jax.experimental.pallas import tpu_sc as plsc

N = 10000
D = 128
H = 16
C = 2
E = 320000

NC = 2
NS = 16
NW = NC * NS
CHUNK = 128
NB = 4
RB = 8
LA = 4
CH = 80
E_PAD = NW * CH * CHUNK
NP = 10240
RPS = NP // NS


def _sc_mesh():
    return plsc.VectorSubcoreMesh(
        core_axis_name="c", subcore_axis_name="s", num_cores=NC, num_subcores=NS
    )


def _deg_body(dstw, out, didx, ones, zbuf, acc, ssem):
    c = lax.axis_index("c")
    s = lax.axis_index("s")
    w = c * NS + s

    def fill_ones(i, _):
        ones[pl.ds(i * 16, 16)] = jnp.ones((16,), jnp.float32)
        return 0

    def fill_zeros(i, _):
        zbuf[pl.ds(i * 16, 16)] = jnp.zeros((16,), jnp.float32)
        return 0

    lax.fori_loop(0, CHUNK // 16, fill_ones, 0)
    lax.fori_loop(0, RPS // 16, fill_zeros, 0)
    pltpu.sync_copy(zbuf, acc.at[pl.ds(s * RPS, RPS)])
    plsc.subcore_barrier()

    pltpu.sync_copy(dstw.at[w], didx)

    def fire(j, b):
        return pltpu.async_copy(ones, acc.at[didx.at[j]], ssem.at[b], add=True)

    def drain(j, b):
        pltpu.make_async_copy(ones, acc.at[didx.at[j]], ssem.at[b]).wait()

    for u in range(NB):
        fire(u, u)

    def group(g, _):
        for u in range(NB):
            j = g * NB + u
            drain(j, u)
            fire(j + NB, u)
        return 0

    lax.fori_loop(0, CH // NB - 1, group, 0)
    for u in range(NB):
        drain((CH // NB - 1) * NB + u, u)
    plsc.subcore_barrier()
    pltpu.sync_copy(acc.at[pl.ds(s * RPS, RPS)], out.at[c, pl.ds(s * RPS, RPS)])


def _agg_body(table, srcw, dstw, out, sidx, didx, rows, zbuf, acc, gsem, ssem):
    c = lax.axis_index("c")
    s = lax.axis_index("s")
    w = c * NS + s

    def zf(i, _):
        zbuf[i] = jnp.zeros((H,), jnp.float32)
        return 0

    lax.fori_loop(0, RPS, zf, 0)
    pltpu.sync_copy(zbuf, acc.at[pl.ds(s * RPS, RPS)])
    plsc.subcore_barrier()

    pltpu.sync_copy(srcw.at[w], sidx)
    pltpu.sync_copy(dstw.at[w], didx)

    def fire_g(j, b):
        return pltpu.async_copy(table.at[sidx.at[j]], rows.at[b], gsem.at[b])

    def wait_g(j, b):
        pltpu.make_async_copy(table.at[sidx.at[j]], rows.at[b], gsem.at[b]).wait()

    def fire_s(j, b):
        return pltpu.async_copy(
            rows.at[b], acc.at[didx.at[j]], ssem.at[b], add=True
        )

    def wait_s(j, b):
        pltpu.make_async_copy(rows.at[b], acc.at[didx.at[j]], ssem.at[b]).wait()

    def step(j, u, do_wait_s):
        wait_g(j, u)
        fire_s(j, u)
        nb = (u + LA) % RB
        if do_wait_s:
            wait_s(j - LA, nb)
        fire_g(j + LA, nb)

    for u in range(LA):
        fire_g(u, u)
    for u in range(RB):
        step(u, u, do_wait_s=u >= LA)

    def group(g, _):
        for u in range(RB):
            j = g * RB + u
            step(j, u, do_wait_s=True)
        return 0

    lax.fori_loop(1, CH // RB - 1, group, 0)

    gl = (CH // RB - 1) * RB
    for u in range(RB):
        j = gl + u
        wait_g(j, u)
        fire_s(j, u)
        if u < LA:
            nb = (u + LA) % RB
            wait_s(j - LA, nb)
            fire_g(j + LA, nb)
    for u in range(RB):
        wait_s(CH - RB + u, u)

    plsc.subcore_barrier()
    pltpu.sync_copy(acc.at[pl.ds(s * RPS, RPS)], out.at[c, pl.ds(s * RPS, RPS)])


@functools.lru_cache(maxsize=1)
def _sc_kernels():
    params = pltpu.CompilerParams(use_tc_tiling_on_sc=False)
    deg = pl.kernel(
        _deg_body,
        out_type=jax.ShapeDtypeStruct((NC, NP), jnp.float32),
        mesh=_sc_mesh(),
        compiler_params=params,
        scratch_types=[
            pltpu.VMEM((CH, CHUNK), jnp.int32),
            pltpu.VMEM((CHUNK,), jnp.float32),
            pltpu.VMEM((RPS,), jnp.float32),
            pltpu.VMEM_SHARED((NP,), jnp.float32),
            pltpu.SemaphoreType.DMA((NB,)),
        ],
    )
    agg = pl.kernel(
        _agg_body,
        out_type=jax.ShapeDtypeStruct((NC, NP, H), jnp.float32),
        mesh=_sc_mesh(),
        compiler_params=params,
        scratch_types=[
            pltpu.VMEM((CH, CHUNK), jnp.int32),
            pltpu.VMEM((CH, CHUNK), jnp.int32),
            pltpu.VMEM((RB, CHUNK, H), jnp.float32),
            pltpu.VMEM((RPS, H), jnp.float32),
            pltpu.VMEM_SHARED((NP, H), jnp.float32),
            pltpu.SemaphoreType.DMA((RB,)),
            pltpu.SemaphoreType.DMA((RB,)),
        ],
    )
    return deg, agg


def _dinv(dp):
    return (1.0 / jnp.sqrt(dp[:, 0:1] + dp[:, 1:2] + 1.0))[:N]


def _tc1_body(x_ref, w1_ref, dp_ref, y1_ref):
    dinv = _dinv(dp_ref[...])
    xw = jnp.dot(x_ref[...], w1_ref[...], preferred_element_type=jnp.float32)
    y1_ref[...] = xw * dinv


def _tc2_body(a0_ref, a1_ref, y1_ref, dp_ref, b1_ref, y2_ref):
    dinv = _dinv(dp_ref[...])
    agg = a0_ref[...][:N] + a1_ref[...][:N] + y1_ref[...]
    h1 = jnp.maximum(agg * dinv + b1_ref[...], 0.0)
    y2_ref[...] = h1 * dinv


def _tc3_body(a0_ref, a1_ref, y2_ref, dp_ref, w2_ref, b2_ref, o_ref):
    dinv = _dinv(dp_ref[...])
    z = (a0_ref[...][:N] + a1_ref[...][:N] + y2_ref[...]) * dinv
    logits = (
        jnp.dot(z, w2_ref[...], preferred_element_type=jnp.float32) + b2_ref[...]
    )
    m = jnp.max(logits, axis=1, keepdims=True)
    lse = m + jnp.log(jnp.sum(jnp.exp(logits - m), axis=1, keepdims=True))
    o_ref[...] = logits - lse


def kernel(x, edge_index, W1, b1, W2, b2):
    src = edge_index[0]
    dst = edge_index[1]
    pad = E_PAD - E
    srcp = jnp.concatenate([src, jnp.zeros((pad,), jnp.int32)]).reshape(
        NW, CH, CHUNK
    )
    dstp = jnp.concatenate([dst, jnp.full((pad,), N, jnp.int32)]).reshape(
        NW, CH, CHUNK
    )

    deg_kernel, agg_kernel = _sc_kernels()
    degp = deg_kernel(dstp)
    dp2 = degp.T

    y1 = pl.pallas_call(
        _tc1_body, out_shape=jax.ShapeDtypeStruct((N, H), jnp.float32)
    )(x, W1, dp2)

    ap1 = agg_kernel(y1, srcp, dstp)
    y2 = pl.pallas_call(
        _tc2_body, out_shape=jax.ShapeDtypeStruct((N, H), jnp.float32)
    )(ap1[0], ap1[1], y1, dp2, b1.reshape(1, H))

    ap2 = agg_kernel(y2, srcp, dstp)
    out = pl.pallas_call(
        _tc3_body, out_shape=jax.ShapeDtypeStruct((N, C), jnp.float32)
    )(ap2[0], ap2[1], y2, dp2, W2, b2.reshape(1, C))
    return out


# speedup vs baseline: 52.7065x; 1.4300x over previous
import functools

import jax
import jax.numpy as jnp
from jax import lax
from jax.experimental import pallas as pl
from jax.experimental.pallas import tpu as pltpu
from jax.experimental.pallas import tpu_sc as plsc

N = 10000
D = 128
H = 16
C = 2
E = 320000

NC = 2
NS = 16
NW = NC * NS
CHUNK = 128
NB = 4
RB = 8
LA = 4
CH = 80
E_PAD = NW * CH * CHUNK
NP = 10240
RPS = NP // NS


def _sc_mesh():
    return plsc.VectorSubcoreMesh(
        core_axis_name="c", subcore_axis_name="s", num_cores=NC, num_subcores=NS
    )


def _deg_body(dstw, out, didx, ones, zbuf, acc, ssem):
    c = lax.axis_index("c")
    s = lax.axis_index("s")
    w = c * NS + s

    def fill_ones(i, _):
        ones[pl.ds(i * 16, 16)] = jnp.ones((16,), jnp.float32)
        return 0

    def fill_zeros(i, _):
        zbuf[pl.ds(i * 16, 16)] = jnp.zeros((16,), jnp.float32)
        return 0

    lax.fori_loop(0, CHUNK // 16, fill_ones, 0)
    lax.fori_loop(0, RPS // 16, fill_zeros, 0)
    pltpu.sync_copy(zbuf, acc.at[pl.ds(s * RPS, RPS)])
    plsc.subcore_barrier()

    pltpu.sync_copy(dstw.at[w], didx)

    def fire(j, b):
        return pltpu.async_copy(ones, acc.at[didx.at[j]], ssem.at[b], add=True)

    def drain(j, b):
        pltpu.make_async_copy(ones, acc.at[didx.at[j]], ssem.at[b]).wait()

    for u in range(NB):
        fire(u, u)

    def group(g, _):
        for u in range(NB):
            j = g * NB + u
            drain(j, u)
            fire(j + NB, u)
        return 0

    lax.fori_loop(0, CH // NB - 1, group, 0)
    for u in range(NB):
        drain((CH // NB - 1) * NB + u, u)
    plsc.subcore_barrier()
    pltpu.sync_copy(acc.at[pl.ds(s * RPS, RPS)], out.at[c, pl.ds(s * RPS, RPS)])


def _agg_body(
    table, srcw, dstw, out, sidx, didx, rows, zbuf, acc, table_s, gsem, ssem
):
    c = lax.axis_index("c")
    s = lax.axis_index("s")
    w = c * NS + s

    def zf(i, _):
        zbuf[i] = jnp.zeros((H,), jnp.float32)
        return 0

    lax.fori_loop(0, RPS, zf, 0)
    pltpu.sync_copy(zbuf, acc.at[pl.ds(s * RPS, RPS)])
    tps = N // NS
    pltpu.sync_copy(
        table.at[pl.ds(s * tps, tps)], table_s.at[pl.ds(s * tps, tps)]
    )
    plsc.subcore_barrier()

    pltpu.sync_copy(srcw.at[w], sidx)
    pltpu.sync_copy(dstw.at[w], didx)

    def fire_g(j, b):
        return pltpu.async_copy(table_s.at[sidx.at[j]], rows.at[b], gsem.at[b])

    def wait_g(j, b):
        pltpu.make_async_copy(
            table_s.at[sidx.at[j]], rows.at[b], gsem.at[b]
        ).wait()

    def fire_s(j, b):
        return pltpu.async_copy(
            rows.at[b], acc.at[didx.at[j]], ssem.at[b], add=True
        )

    def wait_s(j, b):
        pltpu.make_async_copy(rows.at[b], acc.at[didx.at[j]], ssem.at[b]).wait()

    def step(j, u, do_wait_s):
        wait_g(j, u)
        fire_s(j, u)
        nb = (u + LA) % RB
        if do_wait_s:
            wait_s(j - LA, nb)
        fire_g(j + LA, nb)

    for u in range(LA):
        fire_g(u, u)
    for u in range(RB):
        step(u, u, do_wait_s=u >= LA)

    def group(g, _):
        for u in range(RB):
            j = g * RB + u
            step(j, u, do_wait_s=True)
        return 0

    lax.fori_loop(1, CH // RB - 1, group, 0)

    gl = (CH // RB - 1) * RB
    for u in range(RB):
        j = gl + u
        wait_g(j, u)
        fire_s(j, u)
        if u < LA:
            nb = (u + LA) % RB
            wait_s(j - LA, nb)
            fire_g(j + LA, nb)
    for u in range(RB):
        wait_s(CH - RB + u, u)

    plsc.subcore_barrier()
    pltpu.sync_copy(acc.at[pl.ds(s * RPS, RPS)], out.at[c, pl.ds(s * RPS, RPS)])


@functools.lru_cache(maxsize=1)
def _sc_kernels():
    params = pltpu.CompilerParams(use_tc_tiling_on_sc=False)
    deg = pl.kernel(
        _deg_body,
        out_type=jax.ShapeDtypeStruct((NC, NP), jnp.float32),
        mesh=_sc_mesh(),
        compiler_params=params,
        scratch_types=[
            pltpu.VMEM((CH, CHUNK), jnp.int32),
            pltpu.VMEM((CHUNK,), jnp.float32),
            pltpu.VMEM((RPS,), jnp.float32),
            pltpu.VMEM_SHARED((NP,), jnp.float32),
            pltpu.SemaphoreType.DMA((NB,)),
        ],
    )
    agg = pl.kernel(
        _agg_body,
        out_type=jax.ShapeDtypeStruct((NC, NP, H), jnp.float32),
        mesh=_sc_mesh(),
        compiler_params=params,
        scratch_types=[
            pltpu.VMEM((CH, CHUNK), jnp.int32),
            pltpu.VMEM((CH, CHUNK), jnp.int32),
            pltpu.VMEM((RB, CHUNK, H), jnp.float32),
            pltpu.VMEM((RPS, H), jnp.float32),
            pltpu.VMEM_SHARED((NP, H), jnp.float32),
            pltpu.VMEM_SHARED((N, H), jnp.float32),
            pltpu.SemaphoreType.DMA((RB,)),
            pltpu.SemaphoreType.DMA((RB,)),
        ],
    )
    return deg, agg


def _dinv(dp):
    return (1.0 / jnp.sqrt(dp[:, 0:1] + dp[:, 1:2] + 1.0))[:N]


def _tc1_body(x_ref, w1_ref, dp_ref, y1_ref):
    dinv = _dinv(dp_ref[...])
    xw = jnp.dot(x_ref[...], w1_ref[...], preferred_element_type=jnp.float32)
    y1_ref[...] = xw * dinv


def _tc2_body(a0_ref, a1_ref, y1_ref, dp_ref, b1_ref, y2_ref):
    dinv = _dinv(dp_ref[...])
    agg = a0_ref[...][:N] + a1_ref[...][:N] + y1_ref[...]
    h1 = jnp.maximum(agg * dinv + b1_ref[...], 0.0)
    y2_ref[...] = h1 * dinv


def _tc3_body(a0_ref, a1_ref, y2_ref, dp_ref, w2_ref, b2_ref, o_ref):
    dinv = _dinv(dp_ref[...])
    z = (a0_ref[...][:N] + a1_ref[...][:N] + y2_ref[...]) * dinv
    logits = (
        jnp.dot(z, w2_ref[...], preferred_element_type=jnp.float32) + b2_ref[...]
    )
    m = jnp.max(logits, axis=1, keepdims=True)
    lse = m + jnp.log(jnp.sum(jnp.exp(logits - m), axis=1, keepdims=True))
    o_ref[...] = logits - lse


def kernel(x, edge_index, W1, b1, W2, b2):
    src = edge_index[0]
    dst = edge_index[1]
    pad = E_PAD - E
    srcp = jnp.concatenate([src, jnp.zeros((pad,), jnp.int32)]).reshape(
        NW, CH, CHUNK
    )
    dstp = jnp.concatenate([dst, jnp.full((pad,), N, jnp.int32)]).reshape(
        NW, CH, CHUNK
    )

    deg_kernel, agg_kernel = _sc_kernels()
    degp = deg_kernel(dstp)
    dp2 = degp.T

    y1 = pl.pallas_call(
        _tc1_body, out_shape=jax.ShapeDtypeStruct((N, H), jnp.float32)
    )(x, W1, dp2)

    ap1 = agg_kernel(y1, srcp, dstp)
    y2 = pl.pallas_call(
        _tc2_body, out_shape=jax.ShapeDtypeStruct((N, H), jnp.float32)
    )(ap1[0], ap1[1], y1, dp2, b1.reshape(1, H))

    ap2 = agg_kernel(y2, srcp, dstp)
    out = pl.pallas_call(
        _tc3_body, out_shape=jax.ShapeDtypeStruct((N, C), jnp.float32)
    )(ap2[0], ap2[1], y2, dp2, W2, b2.reshape(1, C))
    return out


# speedup vs baseline: 58.5939x; 1.1117x over previous
import functools

import jax
import jax.numpy as jnp
from jax import lax
from jax.experimental import pallas as pl
from jax.experimental.pallas import tpu as pltpu
from jax.experimental.pallas import tpu_sc as plsc

N = 10000
D = 128
H = 16
C = 2
E = 320000

NC = 2
NS = 16
NW = NC * NS
CHUNK = 128
NB = 4
RB = 8
LA = 4
CH = 80
E_PAD = NW * CH * CHUNK
NP = 10240
RPS = NP // NS
L = 16


def _sc_mesh():
    return plsc.VectorSubcoreMesh(
        core_axis_name="c", subcore_axis_name="s", num_cores=NC, num_subcores=NS
    )


def _sc_rsqrt(x):
    i = plsc.bitcast(x, jnp.int32)
    i = jnp.int32(0x5F3759DF) - (i >> 1)
    y = plsc.bitcast(i, jnp.float32)
    for _ in range(3):
        y = y * (1.5 - 0.5 * x * y * y)
    return y


def _load_idx(srcw, dstw, w, sidx, didx):
    pltpu.sync_copy(srcw.at[w], sidx)
    pltpu.sync_copy(dstw.at[w], didx)


def _sweep(table_s, acc, sidx, didx, rows, gsem, ssem):
    def fire_g(j, b):
        return pltpu.async_copy(table_s.at[sidx.at[j]], rows.at[b], gsem.at[b])

    def wait_g(j, b):
        pltpu.make_async_copy(
            table_s.at[sidx.at[j]], rows.at[b], gsem.at[b]
        ).wait()

    def fire_s(j, b):
        return pltpu.async_copy(
            rows.at[b], acc.at[didx.at[j]], ssem.at[b], add=True
        )

    def wait_s(j, b):
        pltpu.make_async_copy(rows.at[b], acc.at[didx.at[j]], ssem.at[b]).wait()

    def step(j, u, do_wait_s):
        wait_g(j, u)
        fire_s(j, u)
        nb = (u + LA) % RB
        if do_wait_s:
            wait_s(j - LA, nb)
        fire_g(j + LA, nb)

    for u in range(LA):
        fire_g(u, u)
    for u in range(RB):
        step(u, u, do_wait_s=u >= LA)

    def group(g, _):
        for u in range(RB):
            step(g * RB + u, u, do_wait_s=True)
        return 0

    lax.fori_loop(1, CH // RB - 1, group, 0)

    gl = (CH // RB - 1) * RB
    for u in range(RB):
        j = gl + u
        wait_g(j, u)
        fire_s(j, u)
        if u < LA:
            nb = (u + LA) % RB
            wait_s(j - LA, nb)
            fire_g(j + LA, nb)
    for u in range(RB):
        wait_s(CH - RB + u, u)


def _deg_body(dstw, out, didx, ones, zbuf, acc, ssem):
    c = lax.axis_index("c")
    s = lax.axis_index("s")
    w = c * NS + s

    def fill_ones(i, _):
        ones[pl.ds(i * L, L)] = jnp.ones((L,), jnp.float32)
        return 0

    def fill_zeros(i, _):
        zbuf[pl.ds(i * L, L)] = jnp.zeros((L,), jnp.float32)
        return 0

    lax.fori_loop(0, CHUNK // L, fill_ones, 0)
    lax.fori_loop(0, RPS // L, fill_zeros, 0)
    pltpu.sync_copy(zbuf, acc.at[pl.ds(s * RPS, RPS)])
    plsc.subcore_barrier()

    pltpu.sync_copy(dstw.at[w], didx)

    def fire(j, b):
        return pltpu.async_copy(ones, acc.at[didx.at[j]], ssem.at[b], add=True)

    def drain(j, b):
        pltpu.make_async_copy(ones, acc.at[didx.at[j]], ssem.at[b]).wait()

    for u in range(NB):
        fire(u, u)

    def group(g, _):
        for u in range(NB):
            j = g * NB + u
            drain(j, u)
            fire(j + NB, u)
        return 0

    lax.fori_loop(0, CH // NB - 1, group, 0)
    for u in range(NB):
        drain((CH // NB - 1) * NB + u, u)
    plsc.subcore_barrier()
    pltpu.sync_copy(acc.at[pl.ds(s * RPS, RPS)], out.at[c, pl.ds(s * RPS, RPS)])


def _stage_dinv(degp, r0, dpbuf):
    pltpu.sync_copy(degp.at[0, pl.ds(r0, RPS)], dpbuf.at[0])
    pltpu.sync_copy(degp.at[1, pl.ds(r0, RPS)], dpbuf.at[1])


def _chunk_dinv(dpbuf, k, dtmp):
    d = dpbuf[0, pl.ds(k * L, L)] + dpbuf[1, pl.ds(k * L, L)] + 1.0
    dtmp[...] = _sc_rsqrt(d)


def _splat(itab, dtmp, u):
    return plsc.load_gather(dtmp, [itab[u]])


def _agg1_body(xwp, degp, idxt, srcw, dstw, out, sidx, didx, rows, xbuf,
               ybuf, dpbuf, dtmp, itab, acc, table_s, gsem, ssem):
    c = lax.axis_index("c")
    s = lax.axis_index("s")
    w = c * NS + s
    r0 = s * RPS

    def zf(i, _):
        ybuf[i] = jnp.zeros((H,), jnp.float32)
        return 0

    lax.fori_loop(0, RPS, zf, 0)
    pltpu.sync_copy(ybuf, acc.at[pl.ds(r0, RPS)])

    _load_idx(srcw, dstw, w, sidx, didx)
    pltpu.sync_copy(xwp.at[pl.ds(r0, RPS)], xbuf)
    pltpu.sync_copy(idxt, itab)
    _stage_dinv(degp, r0, dpbuf)

    def chunk(k, _):
        _chunk_dinv(dpbuf, k, dtmp)
        for u in range(L):
            r = k * L + u
            ybuf[r] = xbuf[r] * _splat(itab, dtmp, u)
        return 0

    lax.fori_loop(0, RPS // L, chunk, 0)
    pltpu.sync_copy(ybuf, table_s.at[pl.ds(r0, RPS)])
    plsc.subcore_barrier()

    _sweep(table_s, acc, sidx, didx, rows, gsem, ssem)

    plsc.subcore_barrier()
    pltpu.sync_copy(acc.at[pl.ds(r0, RPS)], out.at[c, pl.ds(r0, RPS)])


def _agg2_body(xwp, degp, idxt, aparts, b1, srcw, dstw, out, y2out, sidx,
               didx, rows, xbuf, ybuf, a0buf, a1buf, dpbuf, dtmp, itab, b1v,
               acc, table_s, gsem, ssem):
    c = lax.axis_index("c")
    s = lax.axis_index("s")
    w = c * NS + s
    r0 = s * RPS

    def zf(i, _):
        ybuf[i] = jnp.zeros((H,), jnp.float32)
        return 0

    lax.fori_loop(0, RPS, zf, 0)
    pltpu.sync_copy(ybuf, acc.at[pl.ds(r0, RPS)])

    _load_idx(srcw, dstw, w, sidx, didx)
    pltpu.sync_copy(xwp.at[pl.ds(r0, RPS)], xbuf)
    pltpu.sync_copy(aparts.at[0, pl.ds(r0, RPS)], a0buf)
    pltpu.sync_copy(aparts.at[1, pl.ds(r0, RPS)], a1buf)
    pltpu.sync_copy(b1, b1v)
    pltpu.sync_copy(idxt, itab)
    _stage_dinv(degp, r0, dpbuf)
    b1vec = b1v[...]

    def chunk(k, _):
        _chunk_dinv(dpbuf, k, dtmp)
        for u in range(L):
            r = k * L + u
            sp = _splat(itab, dtmp, u)
            t = a0buf[r] + a1buf[r] + xbuf[r] * sp
            t = jnp.maximum(t * sp + b1vec, 0.0)
            ybuf[r] = t * sp
        return 0

    lax.fori_loop(0, RPS // L, chunk, 0)
    pltpu.sync_copy(ybuf, table_s.at[pl.ds(r0, RPS)])

    @pl.when(c == 0)
    def _():
        pltpu.sync_copy(ybuf, y2out.at[pl.ds(r0, RPS)])

    plsc.subcore_barrier()

    _sweep(table_s, acc, sidx, didx, rows, gsem, ssem)

    plsc.subcore_barrier()
    pltpu.sync_copy(acc.at[pl.ds(r0, RPS)], out.at[c, pl.ds(r0, RPS)])


@functools.lru_cache(maxsize=1)
def _sc_kernels():
    params = pltpu.CompilerParams(
        use_tc_tiling_on_sc=False, needs_layout_passes=False
    )
    deg = pl.kernel(
        _deg_body,
        out_type=jax.ShapeDtypeStruct((NC, NP), jnp.float32),
        mesh=_sc_mesh(),
        compiler_params=params,
        scratch_types=[
            pltpu.VMEM((CH, CHUNK), jnp.int32),
            pltpu.VMEM((CHUNK,), jnp.float32),
            pltpu.VMEM((RPS,), jnp.float32),
            pltpu.VMEM_SHARED((NP,), jnp.float32),
            pltpu.SemaphoreType.DMA((NB,)),
        ],
    )
    agg_scratch = [
        pltpu.VMEM((CH, CHUNK), jnp.int32),
        pltpu.VMEM((CH, CHUNK), jnp.int32),
        pltpu.VMEM((RB, CHUNK, H), jnp.float32),
        pltpu.VMEM((RPS, H), jnp.float32),
        pltpu.VMEM((RPS, H), jnp.float32),
    ]
    agg_tail = [
        pltpu.VMEM((2, RPS), jnp.float32),
        pltpu.VMEM((L,), jnp.float32),
        pltpu.VMEM((L, L), jnp.int32),
    ]
    shared_tail = [
        pltpu.VMEM_SHARED((NP, H), jnp.float32),
        pltpu.VMEM_SHARED((NP, H), jnp.float32),
        pltpu.SemaphoreType.DMA((RB,)),
        pltpu.SemaphoreType.DMA((RB,)),
    ]
    agg1 = pl.kernel(
        _agg1_body,
        out_type=jax.ShapeDtypeStruct((NC, NP, H), jnp.float32),
        mesh=_sc_mesh(),
        compiler_params=params,
        scratch_types=agg_scratch + agg_tail + shared_tail,
    )
    agg2 = pl.kernel(
        _agg2_body,
        out_type=(
            jax.ShapeDtypeStruct((NC, NP, H), jnp.float32),
            jax.ShapeDtypeStruct((NP, H), jnp.float32),
        ),
        mesh=_sc_mesh(),
        compiler_params=params,
        scratch_types=agg_scratch
        + [
            pltpu.VMEM((RPS, H), jnp.float32),
            pltpu.VMEM((RPS, H), jnp.float32),
        ]
        + agg_tail
        + [pltpu.VMEM((L,), jnp.float32)]
        + shared_tail,
    )
    return deg, agg1, agg2


def _mm_body(x_ref, w1_ref, o_ref):
    xw = jnp.dot(x_ref[...], w1_ref[...], preferred_element_type=jnp.float32)
    o_ref[:N, :] = xw
    o_ref[N:, :] = jnp.zeros((NP - N, H), jnp.float32)


def _final_body(a0_ref, a1_ref, y2_ref, dp_ref, w2_ref, b2_ref, o_ref):
    dpv = dp_ref[...]
    dinv = (1.0 / jnp.sqrt(dpv[:, 0:1] + dpv[:, 1:2] + 1.0))[:N]
    z = (a0_ref[...][:N] + a1_ref[...][:N] + y2_ref[...][:N]) * dinv
    logits = (
        jnp.dot(z, w2_ref[...], preferred_element_type=jnp.float32) + b2_ref[...]
    )
    m = jnp.max(logits, axis=1, keepdims=True)
    lse = m + jnp.log(jnp.sum(jnp.exp(logits - m), axis=1, keepdims=True))
    o_ref[...] = logits - lse


def kernel(x, edge_index, W1, b1, W2, b2):
    src = edge_index[0]
    dst = edge_index[1]
    pad = E_PAD - E
    srcp = jnp.concatenate([src, jnp.zeros((pad,), jnp.int32)]).reshape(
        NW, CH, CHUNK
    )
    dstp = jnp.concatenate([dst, jnp.full((pad,), N, jnp.int32)]).reshape(
        NW, CH, CHUNK
    )

    deg_kernel, agg1_kernel, agg2_kernel = _sc_kernels()
    degp = deg_kernel(dstp)

    xwp = pl.pallas_call(
        _mm_body, out_shape=jax.ShapeDtypeStruct((NP, H), jnp.float32)
    )(x, W1)

    idxt = jnp.tile(jnp.arange(L, dtype=jnp.int32)[:, None], (1, L))
    ap1 = agg1_kernel(xwp, degp, idxt, srcp, dstp)
    ap2, y2 = agg2_kernel(xwp, degp, idxt, ap1, b1, srcp, dstp)

    out = pl.pallas_call(
        _final_body, out_shape=jax.ShapeDtypeStruct((N, C), jnp.float32)
    )(ap2[0], ap2[1], y2, degp.T, W2, b2.reshape(1, C))
    return out
